# baseline (speedup 1.0000x reference)
import jax
import jax.numpy as jnp
from jax.experimental import pallas as pl

_G = 4


def _cgm_block(x_ref, o_ref):
    xv = x_ref[...]
    m = jnp.max(xv, axis=1, keepdims=True)
    o_ref[...] = jnp.where(xv == m, xv, 0.0)


def kernel(x):
    B, C, W, H = x.shape
    S = W * H
    NG = B * (C // _G)
    x3 = x.reshape(NG, _G, S)
    Gb = 2
    out = pl.pallas_call(
        _cgm_block,
        grid=(NG // Gb,),
        in_specs=[pl.BlockSpec((Gb, _G, S), lambda i: (i, 0, 0))],
        out_specs=pl.BlockSpec((Gb, _G, S), lambda i: (i, 0, 0)),
        out_shape=jax.ShapeDtypeStruct((NG, _G, S), x.dtype),
    )(x3)
    return out.reshape(B, C, W, H)


# speedup vs baseline: 1.9859x; 1.9859x over previous
import jax
import jax.numpy as jnp
from jax.experimental import pallas as pl

_G = 4


def _cgm_block(x_ref, o_ref):
    xv = x_ref[...]
    m = jnp.max(xv, axis=2, keepdims=True)
    o_ref[...] = jnp.where(xv == m, xv, 0.0)


def kernel(x):
    B, C, W, H = x.shape
    NG = C // _G
    x5 = x.reshape(B, NG, _G, W, H)
    out = pl.pallas_call(
        _cgm_block,
        grid=(B, NG),
        in_specs=[pl.BlockSpec((1, 1, _G, W, H), lambda b, g: (b, g, 0, 0, 0))],
        out_specs=pl.BlockSpec((1, 1, _G, W, H), lambda b, g: (b, g, 0, 0, 0)),
        out_shape=jax.ShapeDtypeStruct((B, NG, _G, W, H), x.dtype),
    )(x5)
    return out.reshape(B, C, W, H)
